# baseline (speedup 1.0000x reference)
import functools

import jax
import jax.numpy as jnp
from jax import lax
from jax.experimental import pallas as pl
from jax.experimental.pallas import tpu as pltpu
from jax.experimental.pallas import tpu_sc as plsc

BATCH = 16384
DIM = 32
NC = 2
NS = 16
NW = NC * NS
BPW = BATCH // NW
L = 16
GROUPS = BPW // L


def _body(u_hbm, i_hbm, p_hbm, q_hbm, ub_hbm, ib_hbm, out_hbm,
          idxu_v, idxi_v, rows_p, rows_q, ubv, ibv, s_v,
          sem_p, sem_q, sem_ub, sem_ib):
    wid = lax.axis_index("s") * NC + lax.axis_index("c")
    base = wid * BPW

    pltpu.sync_copy(u_hbm.at[pl.ds(base, BPW)], idxu_v)
    pltpu.sync_copy(i_hbm.at[pl.ds(base, BPW)], idxi_v)

    cp_p = pltpu.async_copy(p_hbm.at[idxu_v], rows_p, sem_p)
    cp_q = pltpu.async_copy(q_hbm.at[idxi_v], rows_q, sem_q)
    cp_ub = pltpu.async_copy(ub_hbm.at[idxu_v], ubv, sem_ub)
    cp_ib = pltpu.async_copy(ib_hbm.at[idxi_v], ibv, sem_ib)
    cp_p.wait()
    cp_q.wait()
    cp_ub.wait()
    cp_ib.wait()

    def group(g, carry):
        row = g * L + lax.iota(jnp.int32, L)
        acc = ubv[pl.ds(g * L, L)] + ibv[pl.ds(g * L, L)]
        for d in range(DIM):
            col = jnp.full((L,), d, jnp.int32)
            pv = plsc.load_gather(rows_p, [row, col])
            qv = plsc.load_gather(rows_q, [row, col])
            acc = acc + pv * qv
        s_v[pl.ds(g * L, L)] = acc
        return carry

    lax.fori_loop(0, GROUPS, group, 0)

    pltpu.sync_copy(s_v, out_hbm.at[pl.ds(base, BPW)])


_mf = functools.partial(
    pl.kernel,
    out_type=jax.ShapeDtypeStruct((BATCH,), jnp.float32),
    mesh=plsc.VectorSubcoreMesh(core_axis_name="c", subcore_axis_name="s"),
    compiler_params=pltpu.CompilerParams(
        needs_layout_passes=False, use_tc_tiling_on_sc=False),
    scratch_types=[
        pltpu.VMEM((BPW,), jnp.int32),
        pltpu.VMEM((BPW,), jnp.int32),
        pltpu.VMEM((BPW, DIM), jnp.float32),
        pltpu.VMEM((BPW, DIM), jnp.float32),
        pltpu.VMEM((BPW,), jnp.float32),
        pltpu.VMEM((BPW,), jnp.float32),
        pltpu.VMEM((BPW,), jnp.float32),
        pltpu.SemaphoreType.DMA,
        pltpu.SemaphoreType.DMA,
        pltpu.SemaphoreType.DMA,
        pltpu.SemaphoreType.DMA,
    ],
)(_body)


def kernel(u, i, P, Q, ub, ib):
    return _mf(u.astype(jnp.int32), i.astype(jnp.int32), P, Q,
               ub.reshape(-1), ib.reshape(-1))
